# baseline (speedup 1.0000x reference)
import functools

import jax
import jax.numpy as jnp
from jax import lax
from jax.experimental import pallas as pl
from jax.experimental.pallas import tpu as pltpu
from jax.experimental.pallas import tpu_sc as plsc

N = 10000
E = 320000
D_IN = 128
D_HID = 64
D_OUT = 40

NP = 10240
NC = 2
NS = 16
NW = NC * NS
CH = 128
NCHT = E // CH
MAXCH = NCHT // NW + 1
NB = 5
RPT = NP // NS
BLK1 = 2048
BLK = 2000

_SC_PARAMS = pltpu.CompilerParams(
    needs_layout_passes=False, use_tc_tiling_on_sc=False)


def _sc_mesh():
    return plsc.VectorSubcoreMesh(core_axis_name="c", subcore_axis_name="s")


@functools.partial(
    pl.kernel,
    out_type=jax.ShapeDtypeStruct((NW, NP), jnp.float32),
    mesh=_sc_mesh(),
    scratch_types=[
        pltpu.VMEM((MAXCH, CH), jnp.int32),
        pltpu.VMEM((NP,), jnp.float32),
    ],
    compiler_params=_SC_PARAMS,
)
def _deg_kernel(edge_hbm, out_hbm, dst_v, hist_v):
    c = lax.axis_index("c")
    s = lax.axis_index("s")
    w = c * NS + s
    lo = (NCHT * w) // NW
    n = (NCHT * (w + 1)) // NW - lo

    def zero_body(i, _):
        hist_v[pl.ds(i * 16, 16)] = jnp.zeros((16,), jnp.float32)
        return ()

    lax.fori_loop(0, NP // 16, zero_body, ())

    pltpu.sync_copy(edge_hbm.at[pl.ds(lo, MAXCH), 1], dst_v)

    ones = jnp.ones((16,), jnp.float32)

    def body(r, _):
        for k in range(CH // 16):
            idx = dst_v[r, pl.ds(k * 16, 16)]
            plsc.addupdate_scatter(hist_v, [idx], ones)
        return ()

    lax.fori_loop(0, n, body, ())
    pltpu.sync_copy(hist_v, out_hbm.at[w])


def _make_msg_kernel(D):
    @functools.partial(
        pl.kernel,
        out_type=jax.ShapeDtypeStruct((NP, 128), jnp.float32),
        mesh=_sc_mesh(),
        scratch_types=[
            pltpu.VMEM((MAXCH, CH), jnp.int32),
            pltpu.VMEM((MAXCH, CH), jnp.int32),
            [pltpu.VMEM((CH, D), jnp.float32) for _ in range(NB)],
            pltpu.VMEM_SHARED((NP, D), jnp.float32),
            pltpu.SemaphoreType.DMA,
        ],
        compiler_params=_SC_PARAMS,
    )
    def msg(edge_hbm, g_hbm, out_hbm, src_v, dst_v, rows, acc_sh, gsem):
        c = lax.axis_index("c")
        s = lax.axis_index("s")
        w = c * NS + s
        lo = (NCHT * w) // NW
        n = (NCHT * (w + 1)) // NW - lo

        pltpu.sync_copy(edge_hbm.at[pl.ds(lo, MAXCH), 0], src_v)
        pltpu.sync_copy(edge_hbm.at[pl.ds(lo, MAXCH), 1], dst_v)

        zoffs = [k * 16 for k in range(D // 16)] + ([D - 16] if D % 16 else [])

        def zrow(r, _):
            for off in zoffs:
                rows[0][r, pl.ds(off, 16)] = jnp.zeros((16,), jnp.float32)
            return ()

        lax.fori_loop(0, CH, zrow, ())
        rbase = pl.multiple_of(s * RPT, 8)

        def zcopy(k, _):
            pltpu.sync_copy(rows[0], acc_sh.at[pl.ds(rbase + k * CH, CH)])
            return ()

        lax.fori_loop(0, RPT // CH, zcopy, ())
        plsc.subcore_barrier()

        for b in range(NB):
            pltpu.async_copy(g_hbm.at[src_v.at[b]], rows[b], gsem)

        full = n // NB

        def outer(j, _):
            for b in range(NB):
                i = j * NB + b
                pltpu.make_async_copy(g_hbm.at[src_v.at[0]], rows[b], gsem).wait()
                pltpu.sync_copy(rows[b], acc_sh.at[dst_v.at[i]], add=True)

                @pl.when(i + NB < n)
                def _():
                    pltpu.async_copy(g_hbm.at[src_v.at[i + NB]], rows[b], gsem)
            return ()

        lax.fori_loop(0, full, outer, ())

        for b in range(NB - 1):
            i = full * NB + b

            @pl.when(i < n)
            def _():
                pltpu.make_async_copy(g_hbm.at[src_v.at[0]], rows[b], gsem).wait()
                pltpu.sync_copy(rows[b], acc_sh.at[dst_v.at[i]], add=True)

        plsc.subcore_barrier()
        cbase_col = pl.multiple_of(c * D, 8)
        pltpu.sync_copy(acc_sh.at[pl.ds(rbase, RPT)],
                        out_hbm.at[pl.ds(rbase, RPT), pl.ds(cbase_col, D)])

    return msg


_msg_hid = _make_msg_kernel(D_HID)
_msg_out = _make_msg_kernel(D_OUT)


def _dinv_block(degp_blk):
    deg = jnp.sum(degp_blk, axis=0) + 1.0
    return lax.rsqrt(deg)[:, None]


_DEGP_SPEC = pl.BlockSpec((NW, BLK1), lambda i: (0, i))


def _tc1a_body(x, w1, h_out):
    h_out[...] = jnp.dot(x[...], w1[...], preferred_element_type=jnp.float32)


_tc1a = pl.pallas_call(
    _tc1a_body,
    grid=(NP // BLK1,),
    in_specs=[
        pl.BlockSpec((BLK1, D_IN), lambda i: (i, 0)),
        pl.BlockSpec((D_IN, D_HID), lambda i: (0, 0)),
    ],
    out_specs=pl.BlockSpec((BLK1, D_HID), lambda i: (i, 0)),
    out_shape=jax.ShapeDtypeStruct((NP, D_HID), jnp.float32),
)


def _tc1b_body(degp, h, g1):
    g1[...] = h[...] * _dinv_block(degp[...])


_tc1b = pl.pallas_call(
    _tc1b_body,
    grid=(NP // BLK1,),
    in_specs=[
        _DEGP_SPEC,
        pl.BlockSpec((BLK1, D_HID), lambda i: (i, 0)),
    ],
    out_specs=pl.BlockSpec((BLK1, D_HID), lambda i: (i, 0)),
    out_shape=jax.ShapeDtypeStruct((NP, D_HID), jnp.float32),
)


def _tc2_body(degp, sp, g1, b1, w2, g2):
    dinv = _dinv_block(degp[...])
    sa = sp[...]
    z = dinv * (sa[:, :D_HID] + sa[:, D_HID:2 * D_HID] + g1[...]) + b1[...]
    h = jnp.maximum(z, 0.0)
    g2[...] = jnp.dot(h, w2[...], preferred_element_type=jnp.float32) * dinv


_tc2 = pl.pallas_call(
    _tc2_body,
    grid=(NP // BLK1,),
    in_specs=[
        _DEGP_SPEC,
        pl.BlockSpec((BLK1, 128), lambda i: (i, 0)),
        pl.BlockSpec((BLK1, D_HID), lambda i: (i, 0)),
        pl.BlockSpec((1, D_HID), lambda i: (0, 0)),
        pl.BlockSpec((D_HID, D_OUT), lambda i: (0, 0)),
    ],
    out_specs=pl.BlockSpec((BLK1, D_OUT), lambda i: (i, 0)),
    out_shape=jax.ShapeDtypeStruct((NP, D_OUT), jnp.float32),
)


def _tc3_body(degp, sp, g2, b2, out):
    dinv = _dinv_block(degp[...])
    sa = sp[...]
    z = dinv * (sa[:, :D_OUT] + sa[:, D_OUT:2 * D_OUT] + g2[...]) + b2[...]
    m = jnp.max(z, axis=1, keepdims=True)
    lse = m + jnp.log(jnp.sum(jnp.exp(z - m), axis=1, keepdims=True))
    out[...] = z - lse


_tc3 = pl.pallas_call(
    _tc3_body,
    grid=(NP // BLK1,),
    in_specs=[
        _DEGP_SPEC,
        pl.BlockSpec((BLK1, 128), lambda i: (i, 0)),
        pl.BlockSpec((BLK1, D_OUT), lambda i: (i, 0)),
        pl.BlockSpec((1, D_OUT), lambda i: (0, 0)),
    ],
    out_specs=pl.BlockSpec((BLK1, D_OUT), lambda i: (i, 0)),
    out_shape=jax.ShapeDtypeStruct((NP, D_OUT), jnp.float32),
)


def kernel(x, edge_index, W1, b1, W2, b2):
    et = edge_index.reshape(2, NCHT, CH).transpose(1, 0, 2)
    x_pad = jnp.pad(x, ((0, NP - N), (0, 0)))

    degp = _deg_kernel(et)
    h1 = _tc1a(x_pad, W1)
    g1 = _tc1b(degp, h1)
    s1 = _msg_hid(et, g1)
    g2 = _tc2(degp, s1, g1, b1.reshape(1, D_HID), W2)
    s2 = _msg_out(et, g2)
    out = _tc3(degp, s2, g2, b2.reshape(1, D_OUT))
    return out[:N]


# speedup vs baseline: 1.0321x; 1.0321x over previous
import functools

import jax
import jax.numpy as jnp
from jax import lax
from jax.experimental import pallas as pl
from jax.experimental.pallas import tpu as pltpu
from jax.experimental.pallas import tpu_sc as plsc

N = 10000
E = 320000
D_IN = 128
D_HID = 64
D_OUT = 40

NP = 10240
NC = 2
NS = 16
NW = NC * NS
CH = 128
NCHT = E // CH
MAXCH = NCHT // NW + 1
NB = 5
RPT = NP // NS
BLK1 = 2048
BLK = 2000

_SC_PARAMS = pltpu.CompilerParams(
    needs_layout_passes=False, use_tc_tiling_on_sc=False)


def _sc_mesh():
    return plsc.VectorSubcoreMesh(core_axis_name="c", subcore_axis_name="s")


@functools.partial(
    pl.kernel,
    out_type=jax.ShapeDtypeStruct((NW, NP), jnp.float32),
    mesh=_sc_mesh(),
    scratch_types=[
        pltpu.VMEM((MAXCH, CH), jnp.int32),
        pltpu.VMEM((NP,), jnp.float32),
    ],
    compiler_params=_SC_PARAMS,
)
def _deg_kernel(edge_hbm, out_hbm, dst_v, hist_v):
    c = lax.axis_index("c")
    s = lax.axis_index("s")
    w = c * NS + s
    lo = (NCHT * w) // NW
    n = (NCHT * (w + 1)) // NW - lo

    def zero_body(i, _):
        hist_v[pl.ds(i * 16, 16)] = jnp.zeros((16,), jnp.float32)
        return ()

    lax.fori_loop(0, NP // 16, zero_body, ())

    pltpu.sync_copy(edge_hbm.at[pl.ds(lo, MAXCH), 1], dst_v)

    ones = jnp.ones((16,), jnp.float32)

    def body(r, _):
        for k in range(CH // 16):
            idx = dst_v[r, pl.ds(k * 16, 16)]
            plsc.addupdate_scatter(hist_v, [idx], ones)
        return ()

    lax.fori_loop(0, n, body, ())
    pltpu.sync_copy(hist_v, out_hbm.at[w])


def _make_msg_kernel(D, double_idx=False):
    @functools.partial(
        pl.kernel,
        out_type=jax.ShapeDtypeStruct((NP, 128), jnp.float32),
        mesh=_sc_mesh(),
        scratch_types=[
            pltpu.VMEM((MAXCH, CH), jnp.int32),
            pltpu.VMEM((MAXCH, CH), jnp.int32),
            [pltpu.VMEM((CH, D), jnp.float32) for _ in range(NB)],
            pltpu.VMEM_SHARED((NP, D), jnp.float32),
            pltpu.SemaphoreType.DMA,
        ],
        compiler_params=_SC_PARAMS,
    )
    def msg(edge_hbm, g_hbm, out_hbm, src_v, dst_v, rows, acc_sh, gsem):
        c = lax.axis_index("c")
        s = lax.axis_index("s")
        w = c * NS + s
        lo = (NCHT * w) // NW
        n = (NCHT * (w + 1)) // NW - lo

        pltpu.sync_copy(edge_hbm.at[pl.ds(lo, MAXCH), 0], src_v)
        pltpu.sync_copy(edge_hbm.at[pl.ds(lo, MAXCH), 1], dst_v)

        def dbl(r, _):
            for k in range(CH // 16):
                sl = pl.ds(k * 16, 16)
                v = src_v[r, sl]
                src_v[r, sl] = v + v
            return ()

        if double_idx:
            lax.fori_loop(0, NB, dbl, ())

        zoffs = [k * 16 for k in range(D // 16)] + ([D - 16] if D % 16 else [])

        def zrow(r, _):
            for off in zoffs:
                rows[0][r, pl.ds(off, 16)] = jnp.zeros((16,), jnp.float32)
            return ()

        lax.fori_loop(0, CH, zrow, ())
        rbase = pl.multiple_of(s * RPT, 8)

        def zcopy(k, _):
            pltpu.sync_copy(rows[0], acc_sh.at[pl.ds(rbase + k * CH, CH)])
            return ()

        lax.fori_loop(0, RPT // CH, zcopy, ())
        plsc.subcore_barrier()

        for b in range(NB):
            pltpu.async_copy(g_hbm.at[src_v.at[b]], rows[b], gsem)

        if double_idx:
            def dbl_rest(r, _):
                return dbl(r, _)

            lax.fori_loop(NB, MAXCH, dbl_rest, ())

        full = n // NB

        def outer(j, _):
            for b in range(NB):
                i = j * NB + b
                pltpu.make_async_copy(g_hbm.at[src_v.at[0]], rows[b], gsem).wait()
                pltpu.sync_copy(rows[b], acc_sh.at[dst_v.at[i]], add=True)

                @pl.when(i + NB < n)
                def _():
                    pltpu.async_copy(g_hbm.at[src_v.at[i + NB]], rows[b], gsem)
            return ()

        lax.fori_loop(0, full, outer, ())

        for b in range(NB - 1):
            i = full * NB + b

            @pl.when(i < n)
            def _():
                pltpu.make_async_copy(g_hbm.at[src_v.at[0]], rows[b], gsem).wait()
                pltpu.sync_copy(rows[b], acc_sh.at[dst_v.at[i]], add=True)

        plsc.subcore_barrier()
        cbase_col = pl.multiple_of(c * D, 8)
        pltpu.sync_copy(acc_sh.at[pl.ds(rbase, RPT)],
                        out_hbm.at[pl.ds(rbase, RPT), pl.ds(cbase_col, D)])

    return msg


_msg_hid = _make_msg_kernel(D_HID, double_idx=True)
_msg_out = _make_msg_kernel(D_OUT)


def _dinv_block(degp_blk):
    deg = jnp.sum(degp_blk, axis=0) + 1.0
    return lax.rsqrt(deg)[:, None]


_DEGP_SPEC = pl.BlockSpec((NW, BLK1), lambda i: (0, i))


def _tc1_body(degp, x, w1, g1):
    dinv = _dinv_block(degp[...])
    h = jnp.dot(x[...], w1[...], preferred_element_type=jnp.float32)
    g1[:, :D_HID] = h * dinv


_tc1 = pl.pallas_call(
    _tc1_body,
    grid=(NP // BLK1,),
    in_specs=[
        _DEGP_SPEC,
        pl.BlockSpec((BLK1, D_IN), lambda i: (i, 0)),
        pl.BlockSpec((D_IN, D_HID), lambda i: (0, 0)),
    ],
    out_specs=pl.BlockSpec((BLK1, 128), lambda i: (i, 0)),
    out_shape=jax.ShapeDtypeStruct((NP, 128), jnp.float32),
)


def _tc2_body(degp, sp, g1, b1, w2, g2):
    dinv = _dinv_block(degp[...])
    sa = sp[...]
    g1a = g1[...]
    z = dinv * (sa[:, :D_HID] + sa[:, D_HID:2 * D_HID] + g1a[:, :D_HID]) + b1[...]
    h = jnp.maximum(z, 0.0)
    g2[...] = jnp.dot(h, w2[...], preferred_element_type=jnp.float32) * dinv


_tc2 = pl.pallas_call(
    _tc2_body,
    grid=(NP // BLK1,),
    in_specs=[
        _DEGP_SPEC,
        pl.BlockSpec((BLK1, 128), lambda i: (i, 0)),
        pl.BlockSpec((BLK1, 128), lambda i: (i, 0)),
        pl.BlockSpec((1, D_HID), lambda i: (0, 0)),
        pl.BlockSpec((D_HID, D_OUT), lambda i: (0, 0)),
    ],
    out_specs=pl.BlockSpec((BLK1, D_OUT), lambda i: (i, 0)),
    out_shape=jax.ShapeDtypeStruct((NP, D_OUT), jnp.float32),
)


def _tc3_body(degp, sp, g2, b2, out):
    dinv = _dinv_block(degp[...])
    sa = sp[...]
    z = dinv * (sa[:, :D_OUT] + sa[:, D_OUT:2 * D_OUT] + g2[...]) + b2[...]
    m = jnp.max(z, axis=1, keepdims=True)
    lse = m + jnp.log(jnp.sum(jnp.exp(z - m), axis=1, keepdims=True))
    out[...] = z - lse


_tc3 = pl.pallas_call(
    _tc3_body,
    grid=(NP // BLK1,),
    in_specs=[
        _DEGP_SPEC,
        pl.BlockSpec((BLK1, 128), lambda i: (i, 0)),
        pl.BlockSpec((BLK1, D_OUT), lambda i: (i, 0)),
        pl.BlockSpec((1, D_OUT), lambda i: (0, 0)),
    ],
    out_specs=pl.BlockSpec((BLK1, D_OUT), lambda i: (i, 0)),
    out_shape=jax.ShapeDtypeStruct((NP, D_OUT), jnp.float32),
)


def kernel(x, edge_index, W1, b1, W2, b2):
    et = edge_index.reshape(2, NCHT, CH).transpose(1, 0, 2)
    x_pad = jnp.pad(x, ((0, NP - N), (0, 0)))

    degp = _deg_kernel(et)
    g1 = _tc1(degp, x_pad, W1)
    s1 = _msg_hid(et, g1.reshape(2 * NP, D_HID))
    g2 = _tc2(degp, s1, g1, b1.reshape(1, D_HID), W2)
    s2 = _msg_out(et, g2)
    out = _tc3(degp, s2, g2, b2.reshape(1, D_OUT))
    return out[:N]


# speedup vs baseline: 1.0472x; 1.0146x over previous
import functools

import jax
import jax.numpy as jnp
from jax import lax
from jax.experimental import pallas as pl
from jax.experimental.pallas import tpu as pltpu
from jax.experimental.pallas import tpu_sc as plsc

N = 10000
E = 320000
D_IN = 128
D_HID = 64
D_OUT = 40

NP = 10240
NC = 2
NS = 16
NW = NC * NS
CH = 128
NCHT = E // CH
MAXCH = NCHT // NW + 1
NB = 5
RPT = NP // NS
BLK1 = 2048
BLK = 2000

_SC_PARAMS = pltpu.CompilerParams(
    needs_layout_passes=False, use_tc_tiling_on_sc=False)


def _sc_mesh():
    return plsc.VectorSubcoreMesh(core_axis_name="c", subcore_axis_name="s")


@functools.partial(
    pl.kernel,
    out_type=jax.ShapeDtypeStruct((NW, NP), jnp.float32),
    mesh=_sc_mesh(),
    scratch_types=[
        pltpu.VMEM((MAXCH, CH), jnp.int32),
        pltpu.VMEM((NP,), jnp.float32),
    ],
    compiler_params=_SC_PARAMS,
)
def _deg_kernel(edge_hbm, out_hbm, dst_v, hist_v):
    c = lax.axis_index("c")
    s = lax.axis_index("s")
    w = c * NS + s
    lo = (NCHT * w) // NW
    n = (NCHT * (w + 1)) // NW - lo

    def zero_body(i, _):
        hist_v[pl.ds(i * 16, 16)] = jnp.zeros((16,), jnp.float32)
        return ()

    lax.fori_loop(0, NP // 16, zero_body, ())

    pltpu.sync_copy(edge_hbm.at[pl.ds(lo, MAXCH), 1], dst_v)

    ones = jnp.ones((16,), jnp.float32)

    def body(r, _):
        for k in range(CH // 16):
            idx = dst_v[r, pl.ds(k * 16, 16)]
            plsc.addupdate_scatter(hist_v, [idx], ones)
        return ()

    lax.fori_loop(0, n, body, ())
    pltpu.sync_copy(hist_v, out_hbm.at[w])


def _make_msg_kernel(D, double_idx=False):
    @functools.partial(
        pl.kernel,
        out_type=jax.ShapeDtypeStruct((NP, 128), jnp.float32),
        mesh=_sc_mesh(),
        scratch_types=[
            pltpu.VMEM((MAXCH, CH), jnp.int32),
            pltpu.VMEM((MAXCH, CH), jnp.int32),
            [pltpu.VMEM((CH, D), jnp.float32) for _ in range(NB)],
            pltpu.VMEM_SHARED((NP, D), jnp.float32),
            pltpu.SemaphoreType.DMA,
        ],
        compiler_params=_SC_PARAMS,
    )
    def msg(edge_hbm, g_hbm, out_hbm, src_v, dst_v, rows, acc_sh, gsem):
        c = lax.axis_index("c")
        s = lax.axis_index("s")
        w = c * NS + s
        lo = (NCHT * w) // NW
        n = (NCHT * (w + 1)) // NW - lo

        pltpu.sync_copy(edge_hbm.at[pl.ds(lo, MAXCH), 0], src_v)
        pltpu.sync_copy(edge_hbm.at[pl.ds(lo, MAXCH), 1], dst_v)

        def dbl(r, _):
            for k in range(CH // 16):
                sl = pl.ds(k * 16, 16)
                v = src_v[r, sl]
                src_v[r, sl] = v + v
            return ()

        if double_idx:
            lax.fori_loop(0, NB, dbl, ())

        zoffs = [k * 16 for k in range(D // 16)] + ([D - 16] if D % 16 else [])

        def zrow(r, _):
            for off in zoffs:
                rows[0][r, pl.ds(off, 16)] = jnp.zeros((16,), jnp.float32)
            return ()

        lax.fori_loop(0, CH, zrow, ())
        rbase = pl.multiple_of(s * RPT, 8)

        def zcopy(k, _):
            pltpu.sync_copy(rows[0], acc_sh.at[pl.ds(rbase + k * CH, CH)])
            return ()

        lax.fori_loop(0, RPT // CH, zcopy, ())
        plsc.subcore_barrier()

        for b in range(NB):
            pltpu.async_copy(g_hbm.at[src_v.at[b]], rows[b], gsem)

        if double_idx:
            def dbl_rest(r, _):
                return dbl(r, _)

            lax.fori_loop(NB, MAXCH, dbl_rest, ())

        full = n // NB

        def outer(j, _):
            for b in range(NB):
                i = j * NB + b
                pltpu.make_async_copy(g_hbm.at[src_v.at[0]], rows[b], gsem).wait()
                pltpu.sync_copy(rows[b], acc_sh.at[dst_v.at[i]], add=True)

                @pl.when(i + NB < n)
                def _():
                    pltpu.async_copy(g_hbm.at[src_v.at[i + NB]], rows[b], gsem)
            return ()

        lax.fori_loop(0, full, outer, ())

        for b in range(NB - 1):
            i = full * NB + b

            @pl.when(i < n)
            def _():
                pltpu.make_async_copy(g_hbm.at[src_v.at[0]], rows[b], gsem).wait()
                pltpu.sync_copy(rows[b], acc_sh.at[dst_v.at[i]], add=True)

        plsc.subcore_barrier()
        cbase_col = pl.multiple_of(c * D, 8)
        pltpu.sync_copy(acc_sh.at[pl.ds(rbase, RPT)],
                        out_hbm.at[pl.ds(rbase, RPT), pl.ds(cbase_col, D)])

    return msg


_msg_hid = _make_msg_kernel(D_HID, double_idx=True)
_msg_out = _make_msg_kernel(D_OUT)


def _dinv_block(degp_blk):
    deg = jnp.sum(degp_blk, axis=0) + 1.0
    return lax.rsqrt(deg)[:, None]


_DEGP_SPEC = pl.BlockSpec((NW, BLK1), lambda i: (0, i))


def _tc1_body(degp, x, w1, g1):
    dinv = _dinv_block(degp[...])
    h = jnp.dot(x[...], w1[...], preferred_element_type=jnp.float32)
    g1[:, :D_HID] = h * dinv


_tc1 = pl.pallas_call(
    _tc1_body,
    grid=(NP // BLK1,),
    in_specs=[
        _DEGP_SPEC,
        pl.BlockSpec((BLK1, D_IN), lambda i: (i, 0)),
        pl.BlockSpec((D_IN, D_HID), lambda i: (0, 0)),
    ],
    out_specs=pl.BlockSpec((BLK1, 128), lambda i: (i, 0)),
    out_shape=jax.ShapeDtypeStruct((N, 128), jnp.float32),
)


def _tc2_body(degp, sp, g1, b1, w2, g2):
    dinv = _dinv_block(degp[...])
    sa = sp[...]
    g1a = g1[...]
    z = dinv * (sa[:, :D_HID] + sa[:, D_HID:2 * D_HID] + g1a[:, :D_HID]) + b1[...]
    h = jnp.maximum(z, 0.0)
    g2[...] = jnp.dot(h, w2[...], preferred_element_type=jnp.float32) * dinv


_tc2 = pl.pallas_call(
    _tc2_body,
    grid=(NP // BLK1,),
    in_specs=[
        _DEGP_SPEC,
        pl.BlockSpec((BLK1, 128), lambda i: (i, 0)),
        pl.BlockSpec((BLK1, 128), lambda i: (i, 0)),
        pl.BlockSpec((1, D_HID), lambda i: (0, 0)),
        pl.BlockSpec((D_HID, D_OUT), lambda i: (0, 0)),
    ],
    out_specs=pl.BlockSpec((BLK1, D_OUT), lambda i: (i, 0)),
    out_shape=jax.ShapeDtypeStruct((N, D_OUT), jnp.float32),
)


def _tc3_body(degp, sp, g2, b2, out):
    dinv = _dinv_block(degp[...])
    sa = sp[...]
    z = dinv * (sa[:, :D_OUT] + sa[:, D_OUT:2 * D_OUT] + g2[...]) + b2[...]
    m = jnp.max(z, axis=1, keepdims=True)
    lse = m + jnp.log(jnp.sum(jnp.exp(z - m), axis=1, keepdims=True))
    out[...] = z - lse


_tc3 = pl.pallas_call(
    _tc3_body,
    grid=(NP // BLK1,),
    in_specs=[
        _DEGP_SPEC,
        pl.BlockSpec((BLK1, 128), lambda i: (i, 0)),
        pl.BlockSpec((BLK1, D_OUT), lambda i: (i, 0)),
        pl.BlockSpec((1, D_OUT), lambda i: (0, 0)),
    ],
    out_specs=pl.BlockSpec((BLK1, D_OUT), lambda i: (i, 0)),
    out_shape=jax.ShapeDtypeStruct((N, D_OUT), jnp.float32),
)


def kernel(x, edge_index, W1, b1, W2, b2):
    et = edge_index.reshape(2, NCHT, CH).transpose(1, 0, 2)

    degp = _deg_kernel(et)
    g1 = _tc1(degp, x, W1)
    s1 = _msg_hid(et, g1.reshape(2 * N, D_HID))
    g2 = _tc2(degp, s1, g1, b1.reshape(1, D_HID), W2)
    s2 = _msg_out(et, g2)
    return _tc3(degp, s2, g2, b2.reshape(1, D_OUT))


# speedup vs baseline: 1.0593x; 1.0115x over previous
import functools

import jax
import jax.numpy as jnp
from jax import lax
from jax.experimental import pallas as pl
from jax.experimental.pallas import tpu as pltpu
from jax.experimental.pallas import tpu_sc as plsc

N = 10000
E = 320000
D_IN = 128
D_HID = 64
D_OUT = 40

NP = 10240
NC = 2
NS = 16
NW = NC * NS
CH = 128
NCHT = E // CH
MAXCH = NCHT // NW + 1
NB = 5
RPT = NP // NS
BLK1 = 2048
BLK = 2000

_SC_PARAMS = pltpu.CompilerParams(
    needs_layout_passes=False, use_tc_tiling_on_sc=False)


def _sc_mesh():
    return plsc.VectorSubcoreMesh(core_axis_name="c", subcore_axis_name="s")


@functools.partial(
    pl.kernel,
    out_type=jax.ShapeDtypeStruct((NW, NP), jnp.float32),
    mesh=_sc_mesh(),
    scratch_types=[
        pltpu.VMEM((MAXCH, CH), jnp.int32),
        pltpu.VMEM((NP,), jnp.float32),
    ],
    compiler_params=_SC_PARAMS,
)
def _deg_kernel(edge_hbm, out_hbm, dst_v, hist_v):
    c = lax.axis_index("c")
    s = lax.axis_index("s")
    w = c * NS + s
    lo = (NCHT * w) // NW
    n = (NCHT * (w + 1)) // NW - lo

    def zero_body(i, _):
        hist_v[pl.ds(i * 16, 16)] = jnp.zeros((16,), jnp.float32)
        return ()

    lax.fori_loop(0, NP // 16, zero_body, ())

    pltpu.sync_copy(edge_hbm.at[pl.ds(lo, MAXCH), 1], dst_v)

    ones = jnp.ones((16,), jnp.float32)

    def body(r, _):
        for k in range(CH // 16):
            idx = dst_v[r, pl.ds(k * 16, 16)]
            plsc.addupdate_scatter(hist_v, [idx], ones)
        return ()

    lax.fori_loop(0, n, body, ())
    pltpu.sync_copy(hist_v, out_hbm.at[w])


def _make_msg_kernel(D, double_idx=False, NB=NB):
    @functools.partial(
        pl.kernel,
        out_type=jax.ShapeDtypeStruct((NP, 128), jnp.float32),
        mesh=_sc_mesh(),
        scratch_types=[
            pltpu.VMEM((MAXCH, CH), jnp.int32),
            pltpu.VMEM((MAXCH, CH), jnp.int32),
            [pltpu.VMEM((CH, D), jnp.float32) for _ in range(NB)],
            pltpu.VMEM_SHARED((NP, D), jnp.float32),
            pltpu.SemaphoreType.DMA,
        ],
        compiler_params=_SC_PARAMS,
    )
    def msg(edge_hbm, g_hbm, out_hbm, src_v, dst_v, rows, acc_sh, gsem):
        c = lax.axis_index("c")
        s = lax.axis_index("s")
        w = c * NS + s
        lo = (NCHT * w) // NW
        n = (NCHT * (w + 1)) // NW - lo

        pltpu.sync_copy(edge_hbm.at[pl.ds(lo, MAXCH), 0], src_v)
        pltpu.sync_copy(edge_hbm.at[pl.ds(lo, MAXCH), 1], dst_v)

        def dbl(r, _):
            for k in range(CH // 16):
                sl = pl.ds(k * 16, 16)
                v = src_v[r, sl]
                src_v[r, sl] = v + v
            return ()

        if double_idx:
            lax.fori_loop(0, NB, dbl, ())

        zoffs = [k * 16 for k in range(D // 16)] + ([D - 16] if D % 16 else [])

        def zrow(r, _):
            for off in zoffs:
                rows[0][r, pl.ds(off, 16)] = jnp.zeros((16,), jnp.float32)
            return ()

        lax.fori_loop(0, CH, zrow, ())
        rbase = pl.multiple_of(s * RPT, 8)

        def zcopy(k, _):
            pltpu.sync_copy(rows[0], acc_sh.at[pl.ds(rbase + k * CH, CH)])
            return ()

        lax.fori_loop(0, RPT // CH, zcopy, ())
        plsc.subcore_barrier()

        for b in range(NB):
            pltpu.async_copy(g_hbm.at[src_v.at[b]], rows[b], gsem)

        if double_idx:
            def dbl_rest(r, _):
                return dbl(r, _)

            lax.fori_loop(NB, MAXCH, dbl_rest, ())

        full = n // NB

        def outer(j, _):
            for b in range(NB):
                i = j * NB + b
                pltpu.make_async_copy(g_hbm.at[src_v.at[0]], rows[b], gsem).wait()
                pltpu.sync_copy(rows[b], acc_sh.at[dst_v.at[i]], add=True)

                @pl.when(i + NB < n)
                def _():
                    pltpu.async_copy(g_hbm.at[src_v.at[i + NB]], rows[b], gsem)
            return ()

        lax.fori_loop(0, full, outer, ())

        for b in range(NB - 1):
            i = full * NB + b

            @pl.when(i < n)
            def _():
                pltpu.make_async_copy(g_hbm.at[src_v.at[0]], rows[b], gsem).wait()
                pltpu.sync_copy(rows[b], acc_sh.at[dst_v.at[i]], add=True)

        plsc.subcore_barrier()
        cbase_col = pl.multiple_of(c * D, 8)
        pltpu.sync_copy(acc_sh.at[pl.ds(rbase, RPT)],
                        out_hbm.at[pl.ds(rbase, RPT), pl.ds(cbase_col, D)])

    return msg


_msg_hid = _make_msg_kernel(D_HID, double_idx=True)
_msg_out = _make_msg_kernel(D_OUT, NB=8)


def _dinv_block(degp_blk):
    deg = jnp.sum(degp_blk, axis=0) + 1.0
    return lax.rsqrt(deg)[:, None]


_DEGP_SPEC = pl.BlockSpec((NW, BLK1), lambda i: (0, i))


def _tc1_body(degp, x, w1, g1):
    dinv = _dinv_block(degp[...])
    h = jnp.dot(x[...], w1[...], preferred_element_type=jnp.float32)
    g1[:, :D_HID] = h * dinv


_tc1 = pl.pallas_call(
    _tc1_body,
    grid=(NP // BLK1,),
    in_specs=[
        _DEGP_SPEC,
        pl.BlockSpec((BLK1, D_IN), lambda i: (i, 0)),
        pl.BlockSpec((D_IN, D_HID), lambda i: (0, 0)),
    ],
    out_specs=pl.BlockSpec((BLK1, 128), lambda i: (i, 0)),
    out_shape=jax.ShapeDtypeStruct((N, 128), jnp.float32),
)


def _tc2_body(degp, sp, g1, b1, w2, g2):
    dinv = _dinv_block(degp[...])
    sa = sp[...]
    g1a = g1[...]
    z = dinv * (sa[:, :D_HID] + sa[:, D_HID:2 * D_HID] + g1a[:, :D_HID]) + b1[...]
    h = jnp.maximum(z, 0.0)
    g2[...] = jnp.dot(h, w2[...], preferred_element_type=jnp.float32) * dinv


_tc2 = pl.pallas_call(
    _tc2_body,
    grid=(NP // BLK1,),
    in_specs=[
        _DEGP_SPEC,
        pl.BlockSpec((BLK1, 128), lambda i: (i, 0)),
        pl.BlockSpec((BLK1, 128), lambda i: (i, 0)),
        pl.BlockSpec((1, D_HID), lambda i: (0, 0)),
        pl.BlockSpec((D_HID, D_OUT), lambda i: (0, 0)),
    ],
    out_specs=pl.BlockSpec((BLK1, D_OUT), lambda i: (i, 0)),
    out_shape=jax.ShapeDtypeStruct((N, D_OUT), jnp.float32),
)


def _tc3_body(degp, sp, g2, b2, out):
    dinv = _dinv_block(degp[...])
    sa = sp[...]
    z = dinv * (sa[:, :D_OUT] + sa[:, D_OUT:2 * D_OUT] + g2[...]) + b2[...]
    m = jnp.max(z, axis=1, keepdims=True)
    lse = m + jnp.log(jnp.sum(jnp.exp(z - m), axis=1, keepdims=True))
    out[...] = z - lse


_tc3 = pl.pallas_call(
    _tc3_body,
    grid=(NP // BLK1,),
    in_specs=[
        _DEGP_SPEC,
        pl.BlockSpec((BLK1, 128), lambda i: (i, 0)),
        pl.BlockSpec((BLK1, D_OUT), lambda i: (i, 0)),
        pl.BlockSpec((1, D_OUT), lambda i: (0, 0)),
    ],
    out_specs=pl.BlockSpec((BLK1, D_OUT), lambda i: (i, 0)),
    out_shape=jax.ShapeDtypeStruct((N, D_OUT), jnp.float32),
)


def kernel(x, edge_index, W1, b1, W2, b2):
    et = edge_index.reshape(2, NCHT, CH).transpose(1, 0, 2)

    degp = _deg_kernel(et)
    g1 = _tc1(degp, x, W1)
    s1 = _msg_hid(et, g1.reshape(2 * N, D_HID))
    g2 = _tc2(degp, s1, g1, b1.reshape(1, D_HID), W2)
    s2 = _msg_out(et, g2)
    return _tc3(degp, s2, g2, b2.reshape(1, D_OUT))
